# Initial kernel scaffold; baseline (speedup 1.0000x reference)
import jax
import jax.numpy as jnp
from jax.experimental import pallas as pl


def kernel(x, edge_index, batch, pos, Wn0, bn0, Ws0, bs0, Wn1, bn1, Ws1, bs1, W_lin, b_lin, W_out, b_out):
    raise NotImplementedError("write your pallas kernel here")


# speedup vs baseline: 5.3094x; 5.3094x over previous
import functools

import jax
import jax.numpy as jnp
from jax import lax
from jax.experimental import pallas as pl
from jax.experimental.pallas import tpu as pltpu
from jax.experimental.pallas import tpu_sc as plsc

N = 10000
E = 320000
D = 128
G = 64

NPAD = 10240
NC = 2
NS = 16
CH = 80
RT = NPAD // NS
PW = 128

_SC_MESH = plsc.VectorSubcoreMesh(core_axis_name="c", subcore_axis_name="s")


@functools.partial(
    pl.kernel, mesh=_SC_MESH,
    out_type=[jax.ShapeDtypeStruct((NPAD, D), jnp.float32),
              jax.ShapeDtypeStruct((NPAD, PW), jnp.float32)],
    scratch_types=[
        pltpu.VMEM((CH,), jnp.int32),
        pltpu.VMEM((CH,), jnp.int32),
        pltpu.VMEM((CH, D), jnp.float32),
        pltpu.VMEM_SHARED((NPAD, D), jnp.float32),
        pltpu.SemaphoreType.DMA,
    ],
    name="sc_segsum_l0")
def _segsum_l0(tab_h, paug_h, src_h, dst_h, outs_h, outp_h,
               src_v, dst_v, rows_v, acc, sem):
  c = lax.axis_index("c")
  s = lax.axis_index("s")
  base_r = s * RT

  def zero_body(i, _):
    for j in range(D // 16):
      rows_v[i, pl.ds(j * 16, 16)] = jnp.zeros((16,), jnp.float32)
    return 0
  lax.fori_loop(0, CH, zero_body, 0)
  for k in range(RT // CH):
    pltpu.sync_copy(rows_v, acc.at[pl.ds(base_r + k * CH, CH)])
  plsc.subcore_barrier()

  ew = E // NS
  base_e = s * ew

  def make_body(table):
    def chunk_body(k, _):
      off = base_e + k * CH
      pltpu.sync_copy(src_h.at[pl.ds(off, CH)], src_v)
      pltpu.sync_copy(dst_h.at[pl.ds(off, CH)], dst_v)
      pltpu.async_copy(table.at[src_v], rows_v, sem).wait()
      pltpu.sync_copy(rows_v, acc.at[dst_v], add=True)
      return 0
    return chunk_body

  @pl.when(c == 0)
  def _():
    lax.fori_loop(0, ew // CH, make_body(tab_h), 0)

  @pl.when(c == 1)
  def _():
    lax.fori_loop(0, ew // CH, make_body(paug_h), 0)

  plsc.subcore_barrier()

  @pl.when(c == 0)
  def _():
    pltpu.sync_copy(acc.at[pl.ds(base_r, RT)], outs_h.at[pl.ds(base_r, RT)])

  @pl.when(c == 1)
  def _():
    pltpu.sync_copy(acc.at[pl.ds(base_r, RT)], outp_h.at[pl.ds(base_r, RT)])


@functools.partial(
    pl.kernel, mesh=_SC_MESH,
    out_type=[jax.ShapeDtypeStruct((NC, NPAD, D), jnp.float32)],
    scratch_types=[
        pltpu.VMEM((CH,), jnp.int32),
        pltpu.VMEM((CH,), jnp.int32),
        pltpu.VMEM((CH, D), jnp.float32),
        pltpu.VMEM_SHARED((NPAD, D), jnp.float32),
        pltpu.SemaphoreType.DMA,
    ],
    name="sc_segsum_l1")
def _segsum_l1(tab_h, src_h, dst_h, outs_h, src_v, dst_v, rows_v, acc, sem):
  c = lax.axis_index("c")
  s = lax.axis_index("s")
  wid = s * NC + c
  base_r = s * RT

  def zero_body(i, _):
    for j in range(D // 16):
      rows_v[i, pl.ds(j * 16, 16)] = jnp.zeros((16,), jnp.float32)
    return 0
  lax.fori_loop(0, CH, zero_body, 0)
  for k in range(RT // CH):
    pltpu.sync_copy(rows_v, acc.at[pl.ds(base_r + k * CH, CH)])
  plsc.subcore_barrier()

  ew = E // (NC * NS)
  base_e = wid * ew

  def chunk_body(k, _):
    off = base_e + k * CH
    pltpu.sync_copy(src_h.at[pl.ds(off, CH)], src_v)
    pltpu.sync_copy(dst_h.at[pl.ds(off, CH)], dst_v)
    pltpu.async_copy(tab_h.at[src_v], rows_v, sem).wait()
    pltpu.sync_copy(rows_v, acc.at[dst_v], add=True)
    return 0
  lax.fori_loop(0, ew // CH, chunk_body, 0)
  plsc.subcore_barrier()

  pltpu.sync_copy(acc.at[pl.ds(base_r, RT)], outs_h.at[c, pl.ds(base_r, RT)])


BLK = 1024


def _sage_block(ssum, psum, paug_blk, h_blk, wnx_ref, wnp_ref, ws_ref, b_ref):
  deg = psum[:, 2:3]
  cd = jnp.maximum(deg, 1.0)
  meanx = ssum / cd
  mr0 = (psum[:, 0:1] - deg * paug_blk[:, 0:1]) / cd
  mr1 = (psum[:, 1:2] - deg * paug_blk[:, 1:2]) / cd
  out = jnp.dot(meanx, wnx_ref[...], preferred_element_type=jnp.float32)
  out += mr0 * wnp_ref[0:1, :] + mr1 * wnp_ref[1:2, :]
  out += jnp.dot(h_blk, ws_ref[...], preferred_element_type=jnp.float32)
  out += b_ref[...]
  return jnp.maximum(out, 0.0)


def _layer0_body(sp_ref, pp_ref, paug_ref, h_ref, wnx_ref, wnp_ref, ws_ref,
                 b_ref, o_ref):
  o_ref[...] = _sage_block(sp_ref[...], pp_ref[...], paug_ref[...], h_ref[...],
                           wnx_ref, wnp_ref, ws_ref, b_ref)


def _final_body(sp_ref, pp_ref, paug_ref, h_ref, wnx_ref, wnp_ref, ws_ref,
                b_ref, bt_ref, wl_ref, bl_ref, wo_ref, bo_ref, o_ref,
                pool_ref):
  b = pl.program_id(0)

  @pl.when(b == 0)
  def _():
    pool_ref[...] = jnp.full((G, D), -jnp.inf, jnp.float32)

  h2 = _sage_block(sp_ref[0] + sp_ref[1], pp_ref[...], paug_ref[...],
                   h_ref[...], wnx_ref, wnp_ref, ws_ref, b_ref)
  bt = bt_ref[...]
  rows = []
  for g in range(G):
    v = jnp.where(bt == g, h2, -jnp.inf)
    rows.append(jnp.max(v, axis=0, keepdims=True))
  pool_ref[...] = jnp.maximum(pool_ref[...], jnp.concatenate(rows, axis=0))

  @pl.when(b == pl.num_programs(0) - 1)
  def _():
    pooled = pool_ref[...]
    pooled = jnp.where(pooled > -jnp.inf, pooled, 0.0)
    z = jnp.dot(pooled, wl_ref[...], preferred_element_type=jnp.float32)
    z = jnp.maximum(z + bl_ref[...], 0.0)
    o_ref[...] = jnp.dot(z, wo_ref[...],
                         preferred_element_type=jnp.float32) + bo_ref[...]


def _row_spec(width):
  return pl.BlockSpec((BLK, width), lambda b: (b, 0))


def _part_spec(width):
  return pl.BlockSpec((NC, BLK, width), lambda b: (0, b, 0))


def _full_spec(shape):
  return pl.BlockSpec(shape, lambda b: tuple(0 for _ in shape))


_GRID = NPAD // BLK

_layer0 = pl.pallas_call(
    _layer0_body,
    grid=(_GRID,),
    in_specs=[
        _row_spec(D), _row_spec(PW), _row_spec(PW), _row_spec(D),
        _full_spec((D, D)), _full_spec((8, D)), _full_spec((D, D)),
        _full_spec((1, D)),
    ],
    out_specs=_row_spec(D),
    out_shape=jax.ShapeDtypeStruct((NPAD, D), jnp.float32),
)

_final = pl.pallas_call(
    _final_body,
    grid=(_GRID,),
    in_specs=[
        _part_spec(D), _row_spec(PW), _row_spec(PW), _row_spec(D),
        _full_spec((D, D)), _full_spec((8, D)), _full_spec((D, D)),
        _full_spec((1, D)),
        _row_spec(1),
        _full_spec((D, D)), _full_spec((1, D)),
        _full_spec((D, 16)), _full_spec((1, 16)),
    ],
    out_specs=_full_spec((G, 16)),
    out_shape=jax.ShapeDtypeStruct((G, 16), jnp.float32),
    scratch_shapes=[pltpu.VMEM((G, D), jnp.float32)],
)


def kernel(x, edge_index, batch, pos, Wn0, bn0, Ws0, bs0, Wn1, bn1, Ws1, bs1,
           W_lin, b_lin, W_out, b_out):
  xpad = jnp.zeros((NPAD, D), jnp.float32).at[:N].set(x)
  paug = (jnp.zeros((NPAD, PW), jnp.float32)
          .at[:N, 0:2].set(pos)
          .at[:N, 2].set(1.0))
  src = edge_index[0]
  dst = edge_index[1]
  batchp = jnp.full((NPAD, 1), G, jnp.int32).at[:N, 0].set(batch)

  def wsplit(Wn):
    wnp = jnp.zeros((8, D), jnp.float32).at[0:2].set(Wn[D:D + 2])
    return Wn[:D], wnp

  wnx0, wnp0 = wsplit(Wn0)
  wnx1, wnp1 = wsplit(Wn1)

  ssum0, psum = _segsum_l0(xpad, paug, src, dst)
  h1 = _layer0(ssum0, psum, paug, xpad, wnx0, wnp0, Ws0,
               (bn0 + bs0).reshape(1, D))
  (sp1,) = _segsum_l1(h1, src, dst)
  out = _final(sp1, psum, paug, h1, wnx1, wnp1, Ws1,
               (bn1 + bs1).reshape(1, D), batchp,
               W_lin, b_lin.reshape(1, D), W_out, b_out.reshape(1, 16))
  return out


# speedup vs baseline: 10.9561x; 2.0635x over previous
import functools

import jax
import jax.numpy as jnp
from jax import lax
from jax.experimental import pallas as pl
from jax.experimental.pallas import tpu as pltpu
from jax.experimental.pallas import tpu_sc as plsc

N = 10000
E = 320000
D = 128
G = 64

NPAD = 10240
NC = 2
NS = 16
CH = 80
KTOT = E // CH
RT = NPAD // NS
PW = 128
K0 = KTOT // NS
K1 = KTOT // (NC * NS)
IB = 50
NBLK = K0 // IB

_SC_MESH = plsc.VectorSubcoreMesh(core_axis_name="c", subcore_axis_name="s")


def _zero_acc_slice(stage_v, acc, base_r):
  def zero_body(i, _):
    for j in range(D // 16):
      stage_v[i, pl.ds(j * 16, 16)] = jnp.zeros((16,), jnp.float32)
    return 0
  lax.fori_loop(0, CH, zero_body, 0)
  for k in range(RT // CH):
    pltpu.sync_copy(stage_v, acc.at[pl.ds(base_r + k * CH, CH)])


def _edge_loop(table, src_all, dst_all, bufs, sems, acc, k_chunks):
  nbuf = len(bufs)

  def idx(ref, k):
    return ref.at[pl.ds(pl.multiple_of(k * CH, CH), CH)]

  for b in range(nbuf):
    pltpu.async_copy(table.at[idx(src_all, b)], bufs[b], sems[b])

  def body(i, _):
    g = i * nbuf
    for b in range(nbuf):
      k = g + b
      pltpu.make_async_copy(table.at[idx(src_all, k)], bufs[b], sems[b]).wait()
      pltpu.sync_copy(bufs[b], acc.at[idx(dst_all, k)], add=True)

      @pl.when(k + nbuf < k_chunks)
      def _():
        pltpu.async_copy(table.at[idx(src_all, k + nbuf)], bufs[b], sems[b])
    return 0
  kc_main = (k_chunks // nbuf) * nbuf
  lax.fori_loop(0, k_chunks // nbuf, body, 0)
  for r in range(kc_main, k_chunks):
    b = r % nbuf
    pltpu.make_async_copy(table.at[idx(src_all, r)], bufs[b], sems[b]).wait()
    pltpu.sync_copy(bufs[b], acc.at[idx(dst_all, r)], add=True)


@functools.partial(
    pl.kernel, mesh=_SC_MESH,
    out_type=[jax.ShapeDtypeStruct((NPAD, D), jnp.float32),
              jax.ShapeDtypeStruct((NPAD, PW), jnp.float32)],
    scratch_types=[
        pltpu.VMEM((IB * CH,), jnp.int32),
        pltpu.VMEM((IB * CH,), jnp.int32),
        pltpu.VMEM((IB * CH,), jnp.int32),
        pltpu.VMEM((IB * CH,), jnp.int32),
        pltpu.VMEM((CH, D), jnp.float32),
        pltpu.VMEM((CH, D), jnp.float32),
        pltpu.VMEM_SHARED((NPAD, D), jnp.float32),
        pltpu.SemaphoreType.DMA,
        pltpu.SemaphoreType.DMA,
        pltpu.SemaphoreType.DMA,
        pltpu.SemaphoreType.DMA,
    ],
    name="sc_segsum_l0")
def _segsum_l0(tab_h, paug_h, src_h, dst_h, outs_h, outp_h,
               srcb0, srcb1, dstb0, dstb1, buf0, buf1, acc,
               sem0, sem1, isem0, isem1):
  c = lax.axis_index("c")
  s = lax.axis_index("s")
  base_r = s * RT
  base_e = s * (K0 * CH)
  iparity = [(srcb0, dstb0, isem0), (srcb1, dstb1, isem1)]

  def load_blk(j, sb, db, sm):
    off = base_e + j * (IB * CH)
    pltpu.async_copy(src_h.at[pl.ds(off, IB * CH)], sb, sm)
    pltpu.async_copy(dst_h.at[pl.ds(off, IB * CH)], db, sm)

  def wait_blk(j, sb, db, sm):
    off = base_e + j * (IB * CH)
    pltpu.make_async_copy(src_h.at[pl.ds(off, IB * CH)], sb, sm).wait()
    pltpu.make_async_copy(dst_h.at[pl.ds(off, IB * CH)], db, sm).wait()

  load_blk(0, *iparity[0])
  _zero_acc_slice(buf0, acc, base_r)
  plsc.subcore_barrier()

  def run(table):
    for j in range(NBLK):
      sb, db, sm = iparity[j % 2]
      wait_blk(j, sb, db, sm)
      if j + 1 < NBLK:
        load_blk(j + 1, *iparity[(j + 1) % 2])
      _edge_loop(table, sb, db, [buf0, buf1], [sem0, sem1], acc, IB)

  @pl.when(c == 0)
  def _():
    run(tab_h)

  @pl.when(c == 1)
  def _():
    run(paug_h)

  plsc.subcore_barrier()

  @pl.when(c == 0)
  def _():
    pltpu.sync_copy(acc.at[pl.ds(base_r, RT)], outs_h.at[pl.ds(base_r, RT)])

  @pl.when(c == 1)
  def _():
    pltpu.sync_copy(acc.at[pl.ds(base_r, RT)], outp_h.at[pl.ds(base_r, RT)])


@functools.partial(
    pl.kernel, mesh=_SC_MESH,
    out_type=[jax.ShapeDtypeStruct((NC, NPAD, D), jnp.float32)],
    scratch_types=[
        pltpu.VMEM((K1 * CH,), jnp.int32),
        pltpu.VMEM((K1 * CH,), jnp.int32),
        pltpu.VMEM((CH, D), jnp.float32),
        pltpu.VMEM((CH, D), jnp.float32),
        pltpu.VMEM_SHARED((NPAD, D), jnp.float32),
        pltpu.SemaphoreType.DMA,
        pltpu.SemaphoreType.DMA,
    ],
    name="sc_segsum_l1")
def _segsum_l1(tab_h, src_h, dst_h, outs_h, src_v, dst_v,
               buf0, buf1, acc, sem0, sem1):
  bufs = [buf0, buf1]
  sems = [sem0, sem1]
  c = lax.axis_index("c")
  s = lax.axis_index("s")
  wid = s * NC + c
  base_r = s * RT

  pltpu.sync_copy(src_h.at[pl.ds(wid * (K1 * CH), K1 * CH)], src_v)
  pltpu.sync_copy(dst_h.at[pl.ds(wid * (K1 * CH), K1 * CH)], dst_v)
  _zero_acc_slice(bufs[0], acc, base_r)
  plsc.subcore_barrier()

  _edge_loop(tab_h, src_v, dst_v, bufs, sems, acc, K1)
  plsc.subcore_barrier()

  pltpu.sync_copy(acc.at[pl.ds(base_r, RT)], outs_h.at[c, pl.ds(base_r, RT)])


BLK = 1024


def _sage_block(ssum, psum, paug_blk, h_blk, wnx_ref, wnp_ref, ws_ref, b_ref):
  deg = psum[:, 2:3]
  cd = jnp.maximum(deg, 1.0)
  meanx = ssum / cd
  mr0 = (psum[:, 0:1] - deg * paug_blk[:, 0:1]) / cd
  mr1 = (psum[:, 1:2] - deg * paug_blk[:, 1:2]) / cd
  out = jnp.dot(meanx, wnx_ref[...], preferred_element_type=jnp.float32)
  out += mr0 * wnp_ref[0:1, :] + mr1 * wnp_ref[1:2, :]
  out += jnp.dot(h_blk, ws_ref[...], preferred_element_type=jnp.float32)
  out += b_ref[...]
  return jnp.maximum(out, 0.0)


def _layer0_body(sp_ref, pp_ref, paug_ref, h_ref, wnx_ref, wnp_ref, ws_ref,
                 b_ref, o_ref):
  o_ref[...] = _sage_block(sp_ref[...], pp_ref[...], paug_ref[...], h_ref[...],
                           wnx_ref, wnp_ref, ws_ref, b_ref)


def _final_body(sp_ref, pp_ref, paug_ref, h_ref, wnx_ref, wnp_ref, ws_ref,
                b_ref, bt_ref, wl_ref, bl_ref, wo_ref, bo_ref, o_ref,
                pool_ref):
  b = pl.program_id(0)

  @pl.when(b == 0)
  def _():
    pool_ref[...] = jnp.full((G, D), -jnp.inf, jnp.float32)

  h2 = _sage_block(sp_ref[0] + sp_ref[1], pp_ref[...], paug_ref[...],
                   h_ref[...], wnx_ref, wnp_ref, ws_ref, b_ref)
  bt = bt_ref[...]
  rows = []
  for g in range(G):
    v = jnp.where(bt == g, h2, -jnp.inf)
    rows.append(jnp.max(v, axis=0, keepdims=True))
  pool_ref[...] = jnp.maximum(pool_ref[...], jnp.concatenate(rows, axis=0))

  @pl.when(b == pl.num_programs(0) - 1)
  def _():
    pooled = pool_ref[...]
    pooled = jnp.where(pooled > -jnp.inf, pooled, 0.0)
    z = jnp.dot(pooled, wl_ref[...], preferred_element_type=jnp.float32)
    z = jnp.maximum(z + bl_ref[...], 0.0)
    o_ref[...] = jnp.dot(z, wo_ref[...],
                         preferred_element_type=jnp.float32) + bo_ref[...]


def _row_spec(width):
  return pl.BlockSpec((BLK, width), lambda b: (b, 0))


def _part_spec(width):
  return pl.BlockSpec((NC, BLK, width), lambda b: (0, b, 0))


def _full_spec(shape):
  return pl.BlockSpec(shape, lambda b: tuple(0 for _ in shape))


_GRID = NPAD // BLK

_layer0 = pl.pallas_call(
    _layer0_body,
    grid=(_GRID,),
    in_specs=[
        _row_spec(D), _row_spec(PW), _row_spec(PW), _row_spec(D),
        _full_spec((D, D)), _full_spec((8, D)), _full_spec((D, D)),
        _full_spec((1, D)),
    ],
    out_specs=_row_spec(D),
    out_shape=jax.ShapeDtypeStruct((NPAD, D), jnp.float32),
)

_final = pl.pallas_call(
    _final_body,
    grid=(_GRID,),
    in_specs=[
        _part_spec(D), _row_spec(PW), _row_spec(PW), _row_spec(D),
        _full_spec((D, D)), _full_spec((8, D)), _full_spec((D, D)),
        _full_spec((1, D)),
        _row_spec(1),
        _full_spec((D, D)), _full_spec((1, D)),
        _full_spec((D, 16)), _full_spec((1, 16)),
    ],
    out_specs=_full_spec((G, 16)),
    out_shape=jax.ShapeDtypeStruct((G, 16), jnp.float32),
    scratch_shapes=[pltpu.VMEM((G, D), jnp.float32)],
)


def kernel(x, edge_index, batch, pos, Wn0, bn0, Ws0, bs0, Wn1, bn1, Ws1, bs1,
           W_lin, b_lin, W_out, b_out):
  xpad = jnp.zeros((NPAD, D), jnp.float32).at[:N].set(x)
  paug = (jnp.zeros((NPAD, PW), jnp.float32)
          .at[:N, 0:2].set(pos)
          .at[:N, 2].set(1.0))
  src = edge_index[0]
  dst = edge_index[1]
  batchp = jnp.full((NPAD, 1), G, jnp.int32).at[:N, 0].set(batch)

  def wsplit(Wn):
    wnp = jnp.zeros((8, D), jnp.float32).at[0:2].set(Wn[D:D + 2])
    return Wn[:D], wnp

  wnx0, wnp0 = wsplit(Wn0)
  wnx1, wnp1 = wsplit(Wn1)

  ssum0, psum = _segsum_l0(xpad, paug, src, dst)
  h1 = _layer0(ssum0, psum, paug, xpad, wnx0, wnp0, Ws0,
               (bn0 + bs0).reshape(1, D))
  (sp1,) = _segsum_l1(h1, src, dst)
  out = _final(sp1, psum, paug, h1, wnx1, wnp1, Ws1,
               (bn1 + bs1).reshape(1, D), batchp,
               W_lin, b_lin.reshape(1, D), W_out, b_out.reshape(1, 16))
  return out


# speedup vs baseline: 13.7082x; 1.2512x over previous
import functools

import jax
import jax.numpy as jnp
from jax import lax
from jax.experimental import pallas as pl
from jax.experimental.pallas import tpu as pltpu
from jax.experimental.pallas import tpu_sc as plsc

N = 10000
E = 320000
D = 128
G = 64

NPAD = 10240
NC = 2
NS = 16
CH = 80
KTOT = E // CH
RT = NPAD // NS
PW = 128
K0 = KTOT // NS
K1 = KTOT // (NC * NS)
IB = 50
NBLK = K0 // IB

_SC_MESH = plsc.VectorSubcoreMesh(core_axis_name="c", subcore_axis_name="s")


def _zero_acc_slice(stage_v, acc, base_r):
  def zero_body(i, _):
    for j in range(D // 16):
      stage_v[i, pl.ds(j * 16, 16)] = jnp.zeros((16,), jnp.float32)
    return 0
  lax.fori_loop(0, CH, zero_body, 0)
  for k in range(RT // CH):
    pltpu.sync_copy(stage_v, acc.at[pl.ds(base_r + k * CH, CH)])


def _edge_loop(table, src_all, dst_all, bufs, sems, acc, k_chunks):
  nbuf = len(bufs)

  def idx(ref, k):
    return ref.at[pl.ds(pl.multiple_of(k * CH, CH), CH)]

  for b in range(nbuf):
    pltpu.async_copy(table.at[idx(src_all, b)], bufs[b], sems[b])

  def body(i, _):
    g = i * nbuf
    for b in range(nbuf):
      k = g + b
      pltpu.make_async_copy(table.at[idx(src_all, k)], bufs[b], sems[b]).wait()
      pltpu.sync_copy(bufs[b], acc.at[idx(dst_all, k)], add=True)

      @pl.when(k + nbuf < k_chunks)
      def _():
        pltpu.async_copy(table.at[idx(src_all, k + nbuf)], bufs[b], sems[b])
    return 0
  kc_main = (k_chunks // nbuf) * nbuf
  lax.fori_loop(0, k_chunks // nbuf, body, 0)
  for r in range(kc_main, k_chunks):
    b = r % nbuf
    pltpu.make_async_copy(table.at[idx(src_all, r)], bufs[b], sems[b]).wait()
    pltpu.sync_copy(bufs[b], acc.at[idx(dst_all, r)], add=True)


@functools.partial(
    pl.kernel, mesh=_SC_MESH,
    out_type=[jax.ShapeDtypeStruct((NPAD, D), jnp.float32),
              jax.ShapeDtypeStruct((NPAD, PW), jnp.float32)],
    scratch_types=[
        pltpu.VMEM((IB * CH,), jnp.int32),
        pltpu.VMEM((IB * CH,), jnp.int32),
        pltpu.VMEM((IB * CH,), jnp.int32),
        pltpu.VMEM((IB * CH,), jnp.int32),
        pltpu.VMEM((CH, D), jnp.float32),
        pltpu.VMEM((CH, D), jnp.float32),
        pltpu.VMEM_SHARED((NPAD, D), jnp.float32),
        pltpu.SemaphoreType.DMA,
        pltpu.SemaphoreType.DMA,
        pltpu.SemaphoreType.DMA,
        pltpu.SemaphoreType.DMA,
    ],
    name="sc_segsum_l0")
def _segsum_l0(tab_h, paug_h, src_h, dst_h, outs_h, outp_h,
               srcb0, srcb1, dstb0, dstb1, buf0, buf1, acc,
               sem0, sem1, isem0, isem1):
  c = lax.axis_index("c")
  s = lax.axis_index("s")
  base_r = s * RT
  base_e = s * (K0 * CH)
  iparity = [(srcb0, dstb0, isem0), (srcb1, dstb1, isem1)]

  def load_blk(j, sb, db, sm):
    off = base_e + j * (IB * CH)
    pltpu.async_copy(src_h.at[pl.ds(off, IB * CH)], sb, sm)
    pltpu.async_copy(dst_h.at[pl.ds(off, IB * CH)], db, sm)

  def wait_blk(j, sb, db, sm):
    off = base_e + j * (IB * CH)
    pltpu.make_async_copy(src_h.at[pl.ds(off, IB * CH)], sb, sm).wait()
    pltpu.make_async_copy(dst_h.at[pl.ds(off, IB * CH)], db, sm).wait()

  load_blk(0, *iparity[0])
  _zero_acc_slice(buf0, acc, base_r)
  plsc.subcore_barrier()

  def run(table):
    for j in range(NBLK):
      sb, db, sm = iparity[j % 2]
      wait_blk(j, sb, db, sm)
      if j + 1 < NBLK:
        load_blk(j + 1, *iparity[(j + 1) % 2])
      _edge_loop(table, sb, db, [buf0, buf1], [sem0, sem1], acc, IB)

  @pl.when(c == 0)
  def _():
    run(tab_h)

  @pl.when(c == 1)
  def _():
    run(paug_h)

  plsc.subcore_barrier()

  @pl.when(c == 0)
  def _():
    pltpu.sync_copy(acc.at[pl.ds(base_r, RT)], outs_h.at[pl.ds(base_r, RT)])

  @pl.when(c == 1)
  def _():
    pltpu.sync_copy(acc.at[pl.ds(base_r, RT)], outp_h.at[pl.ds(base_r, RT)])


@functools.partial(
    pl.kernel, mesh=_SC_MESH,
    out_type=[jax.ShapeDtypeStruct((NC, NPAD, D), jnp.float32)],
    scratch_types=[
        pltpu.VMEM((K1 * CH,), jnp.int32),
        pltpu.VMEM((K1 * CH,), jnp.int32),
        pltpu.VMEM((CH, D), jnp.float32),
        pltpu.VMEM((CH, D), jnp.float32),
        pltpu.VMEM_SHARED((NPAD, D), jnp.float32),
        pltpu.SemaphoreType.DMA,
        pltpu.SemaphoreType.DMA,
    ],
    name="sc_segsum_l1")
def _segsum_l1(tab_h, src_h, dst_h, outs_h, src_v, dst_v,
               buf0, buf1, acc, sem0, sem1):
  bufs = [buf0, buf1]
  sems = [sem0, sem1]
  c = lax.axis_index("c")
  s = lax.axis_index("s")
  wid = s * NC + c
  base_r = s * RT

  pltpu.sync_copy(src_h.at[pl.ds(wid * (K1 * CH), K1 * CH)], src_v)
  pltpu.sync_copy(dst_h.at[pl.ds(wid * (K1 * CH), K1 * CH)], dst_v)
  _zero_acc_slice(bufs[0], acc, base_r)
  plsc.subcore_barrier()

  _edge_loop(tab_h, src_v, dst_v, bufs, sems, acc, K1)
  plsc.subcore_barrier()

  pltpu.sync_copy(acc.at[pl.ds(base_r, RT)], outs_h.at[c, pl.ds(base_r, RT)])


BLK = 1024


def _sage_block(ssum, psum, paug_blk, h_blk, wnx_ref, wnp_ref, ws_ref, b_ref):
  deg = psum[:, 2:3]
  cd = jnp.maximum(deg, 1.0)
  meanx = ssum / cd
  mr0 = (psum[:, 0:1] - deg * paug_blk[:, 0:1]) / cd
  mr1 = (psum[:, 1:2] - deg * paug_blk[:, 1:2]) / cd
  out = jnp.dot(meanx, wnx_ref[...], preferred_element_type=jnp.float32)
  out += mr0 * wnp_ref[0:1, :] + mr1 * wnp_ref[1:2, :]
  out += jnp.dot(h_blk, ws_ref[...], preferred_element_type=jnp.float32)
  out += b_ref[...]
  return jnp.maximum(out, 0.0)


def _layer0_body(sp_ref, pp_ref, paug_ref, h_ref, wnx_ref, wnp_ref, ws_ref,
                 b_ref, o_ref):
  o_ref[...] = _sage_block(sp_ref[...], pp_ref[...], paug_ref[...], h_ref[...],
                           wnx_ref, wnp_ref, ws_ref, b_ref)


def _final_body(sp_ref, pp_ref, paug_ref, h_ref, wnx_ref, wnp_ref, ws_ref,
                b_ref, bt_ref, wl_ref, bl_ref, wo_ref, bo_ref, o_ref,
                pool_ref):
  b = pl.program_id(0)

  @pl.when(b == 0)
  def _():
    pool_ref[...] = jnp.zeros((G, D), jnp.float32)

  h2 = _sage_block(sp_ref[0] + sp_ref[1], pp_ref[...], paug_ref[...],
                   h_ref[...], wnx_ref, wnp_ref, ws_ref, b_ref)
  bt = bt_ref[...]
  g_lo = bt_ref[0, 0]
  g_hi = bt_ref[BLK - 1, 0]
  for g in range(G):
    @pl.when(jnp.logical_and(g >= g_lo, g <= g_hi))
    def _():
      v = jnp.max(jnp.where(bt == g, h2, 0.0), axis=0, keepdims=True)
      pool_ref[g:g + 1, :] = jnp.maximum(pool_ref[g:g + 1, :], v)

  @pl.when(b == pl.num_programs(0) - 1)
  def _():
    pooled = pool_ref[...]
    z = jnp.dot(pooled, wl_ref[...], preferred_element_type=jnp.float32)
    z = jnp.maximum(z + bl_ref[...], 0.0)
    o_ref[...] = jnp.dot(z, wo_ref[...],
                         preferred_element_type=jnp.float32) + bo_ref[...]


def _row_spec(width):
  return pl.BlockSpec((BLK, width), lambda b: (b, 0))


def _part_spec(width):
  return pl.BlockSpec((NC, BLK, width), lambda b: (0, b, 0))


def _full_spec(shape):
  return pl.BlockSpec(shape, lambda b: tuple(0 for _ in shape))


_GRID = NPAD // BLK

_layer0 = pl.pallas_call(
    _layer0_body,
    grid=(_GRID,),
    in_specs=[
        _row_spec(D), _row_spec(PW), _row_spec(PW), _row_spec(D),
        _full_spec((D, D)), _full_spec((8, D)), _full_spec((D, D)),
        _full_spec((1, D)),
    ],
    out_specs=_row_spec(D),
    out_shape=jax.ShapeDtypeStruct((NPAD, D), jnp.float32),
)

_final = pl.pallas_call(
    _final_body,
    grid=(_GRID,),
    in_specs=[
        _part_spec(D), _row_spec(PW), _row_spec(PW), _row_spec(D),
        _full_spec((D, D)), _full_spec((8, D)), _full_spec((D, D)),
        _full_spec((1, D)),
        _row_spec(1),
        _full_spec((D, D)), _full_spec((1, D)),
        _full_spec((D, 16)), _full_spec((1, 16)),
    ],
    out_specs=_full_spec((G, 16)),
    out_shape=jax.ShapeDtypeStruct((G, 16), jnp.float32),
    scratch_shapes=[pltpu.VMEM((G, D), jnp.float32)],
)


def kernel(x, edge_index, batch, pos, Wn0, bn0, Ws0, bs0, Wn1, bn1, Ws1, bs1,
           W_lin, b_lin, W_out, b_out):
  paug = jnp.concatenate(
      [pos, jnp.ones((N, 1), jnp.float32), jnp.zeros((N, PW - 3), jnp.float32)],
      axis=1)
  src = edge_index[0]
  dst = edge_index[1]
  batchp = jnp.pad(batch.reshape(N, 1), ((0, NPAD - N), (0, 0)),
                   constant_values=G)

  def wsplit(Wn):
    return Wn[:D], jnp.pad(Wn[D:D + 2], ((0, 6), (0, 0)))

  wnx0, wnp0 = wsplit(Wn0)
  wnx1, wnp1 = wsplit(Wn1)

  ssum0, psum = _segsum_l0(x, paug, src, dst)
  h1 = _layer0(ssum0, psum, paug, x, wnx0, wnp0, Ws0,
               (bn0 + bs0).reshape(1, D))
  (sp1,) = _segsum_l1(h1, src, dst)
  out = _final(sp1, psum, paug, h1, wnx1, wnp1, Ws1,
               (bn1 + bs1).reshape(1, D), batchp,
               W_lin, b_lin.reshape(1, D), W_out, b_out.reshape(1, 16))
  return out
